# baseline (speedup 1.0000x reference)
import functools

import jax
import jax.numpy as jnp
from jax import lax
from jax.experimental import pallas as pl
from jax.experimental.pallas import tpu as pltpu
from jax.experimental.pallas import tpu_sc as plsc

EMB = 64
HID = 128
P = 3
B = 4096


def _sc_gather3(user_tab, item_tab, users, pos, neg):
    info = plsc.get_sparse_core_info()
    _NC, _NS = info.num_cores, info.num_subcores
    _NW = _NC * _NS
    _BPW = B // _NW
    mesh = plsc.VectorSubcoreMesh(core_axis_name="c", subcore_axis_name="s")

    @functools.partial(
        pl.kernel,
        mesh=mesh,
        out_type=[jax.ShapeDtypeStruct((B, P, HID), jnp.float32)] * 3,
        scratch_types=[
            pltpu.VMEM((_BPW,), jnp.int32),
            pltpu.VMEM((_BPW, P, HID), jnp.float32),
            pltpu.SemaphoreType.DMA,
        ],
    )
    def gather3(utab, itab, u_idx, p_idx, n_idx, u_out, p_out, n_out,
                idx_v, rows_v, sem):
        wid = lax.axis_index("s") * _NC + lax.axis_index("c")
        base = wid * _BPW
        for idx_hbm, tab, out in ((u_idx, utab, u_out),
                                  (p_idx, itab, p_out),
                                  (n_idx, itab, n_out)):
            pltpu.sync_copy(idx_hbm.at[pl.ds(base, _BPW)], idx_v)
            pltpu.async_copy(tab.at[idx_v], rows_v, sem).wait()
            pltpu.sync_copy(rows_v, out.at[pl.ds(base, _BPW)])

    return gather3(user_tab, item_tab, users, pos, neg)


def _max3(a, b, c):
    return jnp.maximum(jnp.maximum(a, b), c)


def _dense_body(u_ref, p_ref, n_ref, wu_ref, wi_ref, a_ref,
                pu_ref, pi_ref, nu_ref, ni_ref):
    wu = wu_ref[...]
    wi = wi_ref[...]
    a = a_ref[...]
    u = u_ref[...]
    proj_u = [jnp.dot(u[:, k, :], wu) for k in range(P)]
    m_tmp = [jnp.dot(x, a) for x in proj_u]
    for i_ref, uo_ref, io_ref in ((p_ref, pu_ref, pi_ref),
                                  (n_ref, nu_ref, ni_ref)):
        iv = i_ref[...]
        proj_i = [jnp.dot(iv[:, k, :], wi) for k in range(P)]
        m = [[jnp.sum(m_tmp[p] * proj_i[q], axis=1, keepdims=True)
              for q in range(P)] for p in range(P)]
        u_logit = [_max3(m[p][0], m[p][1], m[p][2]) for p in range(P)]
        i_logit = [_max3(m[0][q], m[1][q], m[2][q]) for q in range(P)]
        um = _max3(*u_logit)
        ue = [jnp.exp(x - um) for x in u_logit]
        us = ue[0] + ue[1] + ue[2]
        uo_ref[...] = (ue[0] * proj_u[0] + ue[1] * proj_u[1]
                       + ue[2] * proj_u[2]) / us
        im = _max3(*i_logit)
        ie = [jnp.exp(x - im) for x in i_logit]
        isum = ie[0] + ie[1] + ie[2]
        io_ref[...] = (ie[0] * proj_i[0] + ie[1] * proj_i[1]
                       + ie[2] * proj_i[2]) / isum


def _dense_coattention(u_g, p_g, n_g, W_u, W_i, A):
    BT = 512
    row_spec = pl.BlockSpec((BT, P, HID), lambda i: (i, 0, 0))
    full = lambda shape: pl.BlockSpec(shape, lambda i: (0, 0))
    return pl.pallas_call(
        _dense_body,
        grid=(B // BT,),
        in_specs=[row_spec, row_spec, row_spec,
                  full((HID, EMB)), full((HID, EMB)), full((EMB, EMB))],
        out_specs=[pl.BlockSpec((BT, EMB), lambda i: (i, 0))] * 4,
        out_shape=[jax.ShapeDtypeStruct((B, EMB), jnp.float32)] * 4,
    )(u_g, p_g, n_g, W_u, W_i, A)


def kernel(users, pos_items, neg_items, multi_user_embed, multi_item_embed,
           W_u, W_i, A):
    u_g, p_g, n_g = _sc_gather3(
        multi_user_embed, multi_item_embed,
        users.astype(jnp.int32), pos_items.astype(jnp.int32),
        neg_items.astype(jnp.int32))
    pu, pi, nu, ni = _dense_coattention(u_g, p_g, n_g, W_u, W_i, A)
    return (pu, pi, nu, ni)


# speedup vs baseline: 1.9412x; 1.9412x over previous
import functools

import jax
import jax.numpy as jnp
from jax import lax
from jax.experimental import pallas as pl
from jax.experimental.pallas import tpu as pltpu
from jax.experimental.pallas import tpu_sc as plsc

EMB = 64
HID = 128
P = 3
B = 4096


def _sc_gather3(user_tab, item_tab, users, pos, neg):
    info = plsc.get_sparse_core_info()
    _NC, _NS = info.num_cores, info.num_subcores
    _NW = _NC * _NS
    _BPW = B // _NW
    mesh = plsc.VectorSubcoreMesh(core_axis_name="c", subcore_axis_name="s")

    @functools.partial(
        pl.kernel,
        mesh=mesh,
        out_type=[jax.ShapeDtypeStruct((B, P, HID), jnp.float32)] * 3,
        scratch_types=[
            pltpu.VMEM((_BPW,), jnp.int32),
            pltpu.VMEM((_BPW, P, HID), jnp.float32),
            pltpu.SemaphoreType.DMA,
        ],
    )
    def gather3(utab, itab, u_idx, p_idx, n_idx, u_out, p_out, n_out,
                idx_v, rows_v, sem):
        wid = lax.axis_index("s") * _NC + lax.axis_index("c")
        base = wid * _BPW
        for idx_hbm, tab, out in ((u_idx, utab, u_out),
                                  (p_idx, itab, p_out),
                                  (n_idx, itab, n_out)):
            pltpu.sync_copy(idx_hbm.at[pl.ds(base, _BPW)], idx_v)
            pltpu.async_copy(tab.at[idx_v], rows_v, sem).wait()
            pltpu.sync_copy(rows_v, out.at[pl.ds(base, _BPW)])

    return gather3(user_tab, item_tab, users, pos, neg)


def _max3(a, b, c):
    return jnp.maximum(jnp.maximum(a, b), c)


def _dense_body(u_ref, p_ref, n_ref, wu_ref, wi_ref, a_ref,
                pu_ref, pi_ref, nu_ref, ni_ref):
    wu = wu_ref[...]
    wi = wi_ref[...]
    a = a_ref[...]
    u = u_ref[...]
    proj_u = [jnp.dot(u[:, k, :], wu) for k in range(P)]
    m_tmp = [jnp.dot(x, a) for x in proj_u]
    for i_ref, uo_ref, io_ref in ((p_ref, pu_ref, pi_ref),
                                  (n_ref, nu_ref, ni_ref)):
        iv = i_ref[...]
        proj_i = [jnp.dot(iv[:, k, :], wi) for k in range(P)]
        m = [[jnp.sum(m_tmp[p] * proj_i[q], axis=1, keepdims=True)
              for q in range(P)] for p in range(P)]
        u_logit = [_max3(m[p][0], m[p][1], m[p][2]) for p in range(P)]
        i_logit = [_max3(m[0][q], m[1][q], m[2][q]) for q in range(P)]
        um = _max3(*u_logit)
        ue = [jnp.exp(x - um) for x in u_logit]
        us = ue[0] + ue[1] + ue[2]
        uo_ref[...] = (ue[0] * proj_u[0] + ue[1] * proj_u[1]
                       + ue[2] * proj_u[2]) / us
        im = _max3(*i_logit)
        ie = [jnp.exp(x - im) for x in i_logit]
        isum = ie[0] + ie[1] + ie[2]
        io_ref[...] = (ie[0] * proj_i[0] + ie[1] * proj_i[1]
                       + ie[2] * proj_i[2]) / isum


def _dense_coattention(u_g, p_g, n_g, W_u, W_i, A):
    BT = 512
    row_spec = pl.BlockSpec((BT, P, HID), lambda i: (i, 0, 0))
    full = lambda shape: pl.BlockSpec(shape, lambda i: (0, 0))
    return pl.pallas_call(
        _dense_body,
        grid=(B // BT,),
        in_specs=[row_spec, row_spec, row_spec,
                  full((HID, EMB)), full((HID, EMB)), full((EMB, EMB))],
        out_specs=[pl.BlockSpec((BT, EMB), lambda i: (i, 0))] * 4,
        out_shape=[jax.ShapeDtypeStruct((B, EMB), jnp.float32)] * 4,
    )(u_g, p_g, n_g, W_u, W_i, A)


def kernel(users, pos_items, neg_items, multi_user_embed, multi_item_embed,
           W_u, W_i, A):
    u_g = jnp.take(multi_user_embed, users, axis=0)
    p_g = jnp.take(multi_item_embed, pos_items, axis=0)
    n_g = jnp.take(multi_item_embed, neg_items, axis=0)
    pu, pi, nu, ni = _dense_coattention(u_g, p_g, n_g, W_u, W_i, A)
    return (pu, pi, nu, ni)
